# baseline (speedup 1.0000x reference)
import functools

import jax
import jax.numpy as jnp
from jax import lax
from jax.experimental import pallas as pl
from jax.experimental.pallas import tpu as pltpu
from jax.experimental.pallas import tpu_sc as plsc

NUM_EMB = 1000
D = 128
N = 16384

_info = plsc.get_sparse_core_info()
_NC = _info.num_cores
_NS = _info.num_subcores
_L = _info.num_lanes
_NW = _NC * _NS
_BT = N // _NW


def _matvec_body(t_ref, d_ref, o_ref):
    o_ref[...] = jnp.sum(t_ref[...] * d_ref[...], axis=1)


def _row_scores(table, d_row):
    return pl.pallas_call(
        _matvec_body,
        out_shape=jax.ShapeDtypeStruct((NUM_EMB,), jnp.float32),
    )(table, d_row)


_mesh = plsc.VectorSubcoreMesh(core_axis_name="c", subcore_axis_name="s")


@functools.partial(
    pl.kernel,
    mesh=_mesh,
    out_type=jax.ShapeDtypeStruct((N,), jnp.float32),
    scratch_types=[
        pltpu.VMEM((NUM_EMB,), jnp.float32),
        pltpu.VMEM((_BT,), jnp.int32),
        pltpu.VMEM((_BT,), jnp.float32),
        pltpu.SemaphoreType.DMA,
        pltpu.SemaphoreType.DMA,
    ],
    compiler_params=pltpu.CompilerParams(needs_layout_passes=False),
)
def _gather_scores(scores_hbm, ids_hbm, out_hbm,
                   scores_v, ids_v, out_v, sem_s, sem_i):
    wid = lax.axis_index("s") * _NC + lax.axis_index("c")
    base = wid * _BT
    sc_copy = pltpu.async_copy(scores_hbm, scores_v, sem_s)
    id_copy = pltpu.async_copy(ids_hbm.at[pl.ds(base, _BT)], ids_v, sem_i)
    sc_copy.wait()
    id_copy.wait()
    for t in range(_BT // _L):
        idx = ids_v[pl.ds(t * _L, _L)]
        out_v[pl.ds(t * _L, _L)] = plsc.load_gather(scores_v, [idx])
    pltpu.sync_copy(out_v, out_hbm.at[pl.ds(base, _BT)])


def kernel(disease_emb, treatment_ids, treatment_embeddings):
    d_row = disease_emb.reshape(1, D)
    row_scores = _row_scores(treatment_embeddings, d_row)
    ids = treatment_ids.astype(jnp.int32)
    return _gather_scores(row_scores, ids)


# speedup vs baseline: 1.0721x; 1.0721x over previous
import functools

import jax
import jax.numpy as jnp
from jax import lax
from jax.experimental import pallas as pl
from jax.experimental.pallas import tpu as pltpu
from jax.experimental.pallas import tpu_sc as plsc

NUM_EMB = 1000
D = 128
N = 16384

_info = plsc.get_sparse_core_info()
_NC = _info.num_cores
_NS = _info.num_subcores
_L = _info.num_lanes
_NW = _NS
_BT = N // _NW


def _matvec_body(t_ref, d_ref, o_ref):
    o_ref[...] = jnp.sum(t_ref[...] * d_ref[...], axis=1)


def _row_scores(table, d_row):
    return pl.pallas_call(
        _matvec_body,
        out_shape=jax.ShapeDtypeStruct((NUM_EMB,), jnp.float32),
    )(table, d_row)


_mesh = plsc.VectorSubcoreMesh(core_axis_name="c", subcore_axis_name="s", num_cores=1)


@functools.partial(
    pl.kernel,
    mesh=_mesh,
    out_type=jax.ShapeDtypeStruct((N,), jnp.float32),
    scratch_types=[
        pltpu.VMEM((NUM_EMB,), jnp.float32),
        pltpu.VMEM((_BT,), jnp.int32),
        pltpu.VMEM((_BT,), jnp.float32),
        pltpu.SemaphoreType.DMA,
        pltpu.SemaphoreType.DMA,
    ],
    compiler_params=pltpu.CompilerParams(needs_layout_passes=False),
)
def _gather_scores(scores_hbm, ids_hbm, out_hbm,
                   scores_v, ids_v, out_v, sem_s, sem_i):
    wid = lax.axis_index("s")
    base = wid * _BT
    sc_copy = pltpu.async_copy(scores_hbm, scores_v, sem_s)
    id_copy = pltpu.async_copy(ids_hbm.at[pl.ds(base, _BT)], ids_v, sem_i)
    sc_copy.wait()
    id_copy.wait()
    for t in range(_BT // _L):
        idx = ids_v[pl.ds(t * _L, _L)]
        out_v[pl.ds(t * _L, _L)] = plsc.load_gather(scores_v, [idx])
    pltpu.sync_copy(out_v, out_hbm.at[pl.ds(base, _BT)])


def kernel(disease_emb, treatment_ids, treatment_embeddings):
    d_row = disease_emb.reshape(1, D)
    row_scores = _row_scores(treatment_embeddings, d_row)
    ids = treatment_ids.astype(jnp.int32)
    return _gather_scores(row_scores, ids)


# speedup vs baseline: 1.0958x; 1.0221x over previous
import functools

import jax
import jax.numpy as jnp
from jax import lax
from jax.experimental import pallas as pl
from jax.experimental.pallas import tpu as pltpu
from jax.experimental.pallas import tpu_sc as plsc

NUM_EMB = 1000
D = 128
N = 16384

_info = plsc.get_sparse_core_info()
_NC = _info.num_cores
_NS = _info.num_subcores
_L = _info.num_lanes
_NW = _NS
_BT = N // _NW


def _matvec_body(t_ref, d_ref, o_ref):
    o_ref[...] = jnp.sum(t_ref[...] * d_ref[...], axis=1)


def _row_scores(table, d_row):
    return pl.pallas_call(
        _matvec_body,
        out_shape=jax.ShapeDtypeStruct((NUM_EMB,), jnp.float32),
    )(table, d_row)


_mesh = plsc.VectorSubcoreMesh(core_axis_name="c", subcore_axis_name="s", num_cores=1)


@functools.partial(
    pl.kernel,
    mesh=_mesh,
    out_type=jax.ShapeDtypeStruct((N,), jnp.float32),
    scratch_types=[
        pltpu.VMEM((NUM_EMB,), jnp.float32),
        pltpu.VMEM((_BT,), jnp.int32),
        pltpu.VMEM((_BT,), jnp.float32),
        pltpu.SemaphoreType.DMA,
        pltpu.SemaphoreType.DMA,
    ],
    compiler_params=pltpu.CompilerParams(needs_layout_passes=False),
)
def _gather_scores(scores_hbm, ids_hbm, out_hbm,
                   scores_v, ids_v, out_v, sem_s, sem_i):
    wid = lax.axis_index("s")
    base = wid * _BT
    sc_copy = pltpu.async_copy(scores_hbm, scores_v, sem_s)
    id_copy = pltpu.async_copy(ids_hbm.at[pl.ds(base, _BT)], ids_v, sem_i)
    sc_copy.wait()
    id_copy.wait()
    def _gather_step(t, carry):
        off = pl.multiple_of(t * _L, _L)
        idx = ids_v[pl.ds(off, _L)]
        out_v[pl.ds(off, _L)] = plsc.load_gather(scores_v, [idx])
        return carry
    lax.fori_loop(0, _BT // _L, _gather_step, 0)
    pltpu.sync_copy(out_v, out_hbm.at[pl.ds(base, _BT)])


def kernel(disease_emb, treatment_ids, treatment_embeddings):
    d_row = disease_emb.reshape(1, D)
    row_scores = _row_scores(treatment_embeddings, d_row)
    ids = treatment_ids.astype(jnp.int32)
    return _gather_scores(row_scores, ids)
